# Initial kernel scaffold; baseline (speedup 1.0000x reference)
import jax
import jax.numpy as jnp
from jax.experimental import pallas as pl


def kernel(pin_static, pin_dyn_anchor, d_anchor, edge_src, edge_dst, edge_type, topo_order, node_level, data_mask, edge_valid, source_mask, endpoint_ids, rat_true, z_cont, process_id, edge_cell_type_src, edge_cell_type_dst, edge_pin_role_src, edge_pin_role_dst, edge_fanin_src, edge_fanout_src, edge_fanin_dst, edge_fanout_dst, edge_cap_src, edge_cap_dst, edge_scalars_normed, process_embed, pvt_proc_embed, vW, vb, tW, tb, Ws0, Wn0, b0, Ws1, Wn1, b1, Ws2, Wn2, b2, cell_embed, role_embed, type_embed, eW1, eb1, eW2, eb2, aW, ab, sW, sb):
    raise NotImplementedError("write your pallas kernel here")


# speedup vs baseline: 1.1389x; 1.1389x over previous
import functools
import jax
import jax.numpy as jnp
from jax.experimental import pallas as pl
from jax.experimental.pallas import tpu as pltpu

_N = 10000
_E = 320000
_K = 3
_H = 128
_P = 1000
_LMAX = 8

_BN = 1024
_NP = 10240
_BE = 2000
_NEG = jnp.float32(-1e9)


def _gnn_layer_body(h_ref, acc_ref, deg_ref, ws_ref, wn_ref, b_ref, o_ref, *, res):
    h = h_ref[...]
    neigh = acc_ref[...] / deg_ref[...]
    x = jnp.dot(h, ws_ref[...], preferred_element_type=jnp.float32)
    x = x + jnp.dot(neigh, wn_ref[...], preferred_element_type=jnp.float32)
    x = x + b_ref[...]
    m = jnp.mean(x, axis=-1, keepdims=True)
    v = jnp.mean((x - m) * (x - m), axis=-1, keepdims=True)
    y = jax.nn.relu((x - m) / jnp.sqrt(v + 1e-5))
    if res:
        y = 0.5 * y + 0.5 * h
    o_ref[...] = y


def _gnn_layer(h, acc, deg2, Ws, Wn, b, res):
    np_, din = h.shape
    return pl.pallas_call(
        functools.partial(_gnn_layer_body, res=res),
        grid=(np_ // _BN,),
        in_specs=[
            pl.BlockSpec((_BN, din), lambda i: (i, 0)),
            pl.BlockSpec((_BN, din), lambda i: (i, 0)),
            pl.BlockSpec((_BN, 1), lambda i: (i, 0)),
            pl.BlockSpec((din, _H), lambda i: (0, 0)),
            pl.BlockSpec((din, _H), lambda i: (0, 0)),
            pl.BlockSpec((1, _H), lambda i: (0, 0)),
        ],
        out_specs=pl.BlockSpec((_BN, _H), lambda i: (i, 0)),
        out_shape=jax.ShapeDtypeStruct((np_, _H), jnp.float32),
    )(h, acc, deg2, Ws, Wn, b)


def _edge_head_body(ein_ref, dat_ref, w1_ref, b1_ref, w2_ref, b2_ref,
                    awt_ref, ca_ref, swt_ref, cs_ref,
                    dhat_ref, ge_ref, sh_ref, ls_ref):
    x = jnp.dot(ein_ref[...], w1_ref[...], preferred_element_type=jnp.float32)
    x = jax.nn.relu(x + b1_ref[...])
    e = jnp.dot(x, w2_ref[...], preferred_element_type=jnp.float32)
    e = jax.nn.relu(e + b2_ref[...])
    ca = ca_ref[...]
    awt = awt_ref[...]
    l0 = jnp.sum(e * awt[0:1, :], axis=-1, keepdims=True) + ca[:, 0:1]
    l1 = jnp.sum(e * awt[1:2, :], axis=-1, keepdims=True) + ca[:, 1:2]
    l2 = jnp.sum(e * awt[2:3, :], axis=-1, keepdims=True) + ca[:, 2:3]
    m = jnp.maximum(jnp.maximum(l0, l1), l2)
    e0 = jnp.exp(l0 - m)
    e1 = jnp.exp(l1 - m)
    e2 = jnp.exp(l2 - m)
    inv = 1.0 / (e0 + e1 + e2)
    s0 = e0 * inv
    s1 = e1 * inv
    s2 = e2 * inv
    ls = jnp.sum(e * swt_ref[...], axis=-1, keepdims=True) + cs_ref[:, 0:1]
    ls = jnp.clip(ls, -3.0, 3.0)
    da = dat_ref[...]
    ge = (s0 * da[:, 0:1] + s1 * da[:, 1:2] + s2 * da[:, 2:3]) * jnp.exp(ls)
    dhat_ref[...] = jax.nn.relu(ge)
    ge_ref[...] = ge
    sh_ref[:, 0:1] = s0
    sh_ref[:, 1:2] = s1
    sh_ref[:, 2:3] = s2
    ls_ref[...] = ls


def _edge_head(ein, dat, w1, b1, w2, b2, awt, ca, swt, cs):
    ein_w = ein.shape[1]
    grid = _E // _BE
    outs = [
        jax.ShapeDtypeStruct((_E, 1), jnp.float32),
        jax.ShapeDtypeStruct((_E, 1), jnp.float32),
        jax.ShapeDtypeStruct((_E, _K), jnp.float32),
        jax.ShapeDtypeStruct((_E, 1), jnp.float32),
    ]
    return pl.pallas_call(
        _edge_head_body,
        grid=(grid,),
        in_specs=[
            pl.BlockSpec((_BE, ein_w), lambda i: (i, 0)),
            pl.BlockSpec((_BE, _K), lambda i: (i, 0)),
            pl.BlockSpec((ein_w, _H), lambda i: (0, 0)),
            pl.BlockSpec((1, _H), lambda i: (0, 0)),
            pl.BlockSpec((_H, _H), lambda i: (0, 0)),
            pl.BlockSpec((1, _H), lambda i: (0, 0)),
            pl.BlockSpec((_K, _H), lambda i: (0, 0)),
            pl.BlockSpec((1, _K), lambda i: (0, 0)),
            pl.BlockSpec((1, _H), lambda i: (0, 0)),
            pl.BlockSpec((1, 1), lambda i: (0, 0)),
        ],
        out_specs=[
            pl.BlockSpec((_BE, 1), lambda i: (i, 0)),
            pl.BlockSpec((_BE, 1), lambda i: (i, 0)),
            pl.BlockSpec((_BE, _K), lambda i: (i, 0)),
            pl.BlockSpec((_BE, 1), lambda i: (i, 0)),
        ],
        out_shape=outs,
    )(ein, dat, w1, b1, w2, b2, awt, ca, swt, cs)


def kernel(pin_static, pin_dyn_anchor, d_anchor, edge_src, edge_dst, edge_type,
           topo_order, node_level, data_mask, edge_valid, source_mask,
           endpoint_ids, rat_true, z_cont, process_id, edge_cell_type_src,
           edge_cell_type_dst, edge_pin_role_src, edge_pin_role_dst,
           edge_fanin_src, edge_fanout_src, edge_fanin_dst, edge_fanout_dst,
           edge_cap_src, edge_cap_dst, edge_scalars_normed, process_embed,
           pvt_proc_embed, vW, vb, tW, tb, Ws0, Wn0, b0, Ws1, Wn1, b1,
           Ws2, Wn2, b2, cell_embed, role_embed, type_embed, eW1, eb1,
           eW2, eb2, aW, ab, sW, sb):
    pid = process_id[0]
    proc_emb = process_embed[pid]
    z_t = jnp.concatenate([proc_emb, z_cont], axis=-1)
    z_pvt = pvt_proc_embed[pid] + z_cont[2:3] @ vW + vb + z_cont[3:4] @ tW + tb

    pin_dyn_flat = jnp.transpose(pin_dyn_anchor, (1, 0, 2)).reshape(_N, _K * 4)
    node_in = jnp.concatenate(
        [pin_static, pin_dyn_flat,
         jnp.broadcast_to(z_t[None, :], (_N, z_t.shape[0]))], axis=-1)
    din0 = 32
    node_in = jnp.pad(node_in, ((0, _NP - _N), (0, din0 - node_in.shape[1])))

    deg = jnp.clip(jnp.zeros((_N,), jnp.float32).at[edge_dst].add(1.0), 1.0, None)
    deg2 = jnp.pad(deg, (0, _NP - _N), constant_values=1.0)[:, None]

    Ws0p = jnp.pad(Ws0, ((0, din0 - Ws0.shape[0]), (0, 0)))
    Wn0p = jnp.pad(Wn0, ((0, din0 - Wn0.shape[0]), (0, 0)))

    h = node_in
    for (Ws, Wn, b, res) in ((Ws0p, Wn0p, b0, False), (Ws1, Wn1, b1, True),
                             (Ws2, Wn2, b2, True)):
        acc = jnp.zeros((_NP, h.shape[1]), jnp.float32).at[edge_dst].add(h[edge_src])
        h = _gnn_layer(h, acc, deg2, Ws, Wn, b[None, :], res)

    hn = h[:_N]
    ein = jnp.concatenate(
        [hn[edge_src], hn[edge_dst],
         cell_embed[edge_cell_type_src], cell_embed[edge_cell_type_dst],
         role_embed[edge_pin_role_src], role_embed[edge_pin_role_dst],
         type_embed[edge_type], edge_scalars_normed], axis=-1)
    ein_w = 384
    ein = jnp.pad(ein, ((0, 0), (0, ein_w - ein.shape[1])))
    eW1p = jnp.pad(eW1, ((0, ein_w - eW1.shape[0]), (0, 0)))

    ca = (z_pvt @ aW[_H:] + ab)[None, :]
    cs = (z_pvt @ sW[_H:] + sb)[None, :]
    awt = jnp.transpose(aW[:_H])
    swt = jnp.transpose(sW[:_H])
    dat = jnp.transpose(d_anchor)

    dhat2, ge2, s_hat, ls2 = _edge_head(
        ein, dat, eW1p, eb1[None, :], eW2, eb2[None, :], awt, ca, swt, cs)
    d_hat = dhat2[:, 0]
    g_e = ge2[:, 0]
    log_scale = ls2[:, 0]
    gG = jnp.mean(g_e)

    at = jnp.where(source_mask, jnp.float32(0.0), _NEG)
    for lvl in range(1, _LMAX):
        msgs = jnp.where(edge_valid, at[edge_src] + d_hat, _NEG)
        cand = jnp.full((_N,), _NEG, jnp.float32).at[edge_dst].max(msgs)
        at = jnp.where(node_level == lvl, jnp.maximum(at, cand), at)

    at_ep = at[endpoint_ids]
    slack_hat = rat_true - at_ep
    return d_hat, at, at_ep, slack_hat, g_e, gG, s_hat, log_scale


# speedup vs baseline: 5.0187x; 4.4066x over previous
import functools
import jax
import jax.numpy as jnp
from jax import lax
from jax.experimental import pallas as pl
from jax.experimental.pallas import tpu as pltpu
from jax.experimental.pallas import tpu_sc as plsc

_N = 10000
_E = 320000
_K = 3
_H = 128
_P = 1000
_LMAX = 8

_NP = 10240
_BN = 1024
_BE = 2560
_EPT = _E // 32
_NEG = -1e9

_DEG_COL = 26


def _gnn0_body(h_ref, a0_ref, a1_ref, ws_ref, wn_ref, b_ref, o_ref, deg_ref):
    h = h_ref[...]
    acc = a0_ref[...] + a1_ref[...]
    deg = jnp.clip(acc[:, _DEG_COL:_DEG_COL + 1], 1.0, None)
    neigh = acc / deg
    x = jnp.dot(h, ws_ref[...], preferred_element_type=jnp.float32)
    x = x + jnp.dot(neigh, wn_ref[...], preferred_element_type=jnp.float32)
    x = x + b_ref[...]
    m = jnp.mean(x, axis=-1, keepdims=True)
    v = jnp.mean((x - m) * (x - m), axis=-1, keepdims=True)
    o_ref[...] = jax.nn.relu((x - m) / jnp.sqrt(v + 1e-5))
    deg_ref[...] = deg


def _gnn0(h, accflat, Ws, Wn, b):
    din = h.shape[1]
    nblk = _NP // _BN
    return pl.pallas_call(
        _gnn0_body,
        grid=(nblk,),
        in_specs=[
            pl.BlockSpec((_BN, din), lambda i: (i, 0)),
            pl.BlockSpec((_BN, din), lambda i: (i, 0)),
            pl.BlockSpec((_BN, din), lambda i: (i + _NP // _BN, 0)),
            pl.BlockSpec((din, _H), lambda i: (0, 0)),
            pl.BlockSpec((din, _H), lambda i: (0, 0)),
            pl.BlockSpec((1, _H), lambda i: (0, 0)),
        ],
        out_specs=[
            pl.BlockSpec((_BN, _H), lambda i: (i, 0)),
            pl.BlockSpec((_BN, 1), lambda i: (i, 0)),
        ],
        out_shape=[
            jax.ShapeDtypeStruct((_NP, _H), jnp.float32),
            jax.ShapeDtypeStruct((_NP, 1), jnp.float32),
        ],
    )(h, accflat, accflat, Ws, Wn, b)


def _gnn12_body(h_ref, a0_ref, a1_ref, deg_ref, ws_ref, wn_ref, b_ref, o_ref):
    h = h_ref[...]
    neigh = (a0_ref[...] + a1_ref[...]) / deg_ref[...]
    x = jnp.dot(h, ws_ref[...], preferred_element_type=jnp.float32)
    x = x + jnp.dot(neigh, wn_ref[...], preferred_element_type=jnp.float32)
    x = x + b_ref[...]
    m = jnp.mean(x, axis=-1, keepdims=True)
    v = jnp.mean((x - m) * (x - m), axis=-1, keepdims=True)
    y = jax.nn.relu((x - m) / jnp.sqrt(v + 1e-5))
    o_ref[...] = 0.5 * y + 0.5 * h


def _gnn12(h, accflat, deg2, Ws, Wn, b):
    nblk = _NP // _BN
    return pl.pallas_call(
        _gnn12_body,
        grid=(nblk,),
        in_specs=[
            pl.BlockSpec((_BN, _H), lambda i: (i, 0)),
            pl.BlockSpec((_BN, _H), lambda i: (i, 0)),
            pl.BlockSpec((_BN, _H), lambda i: (i + _NP // _BN, 0)),
            pl.BlockSpec((_BN, 1), lambda i: (i, 0)),
            pl.BlockSpec((_H, _H), lambda i: (0, 0)),
            pl.BlockSpec((_H, _H), lambda i: (0, 0)),
            pl.BlockSpec((1, _H), lambda i: (0, 0)),
        ],
        out_specs=pl.BlockSpec((_BN, _H), lambda i: (i, 0)),
        out_shape=jax.ShapeDtypeStruct((_NP, _H), jnp.float32),
    )(h, accflat, accflat, deg2, Ws, Wn, b)


def _make_seg_sum(D):
    mesh = plsc.VectorSubcoreMesh(core_axis_name="c", subcore_axis_name="s")
    rows_per_tile = _NP // 16

    @functools.partial(
        pl.kernel,
        out_type=jax.ShapeDtypeStruct((2 * _NP, D), jnp.float32),
        mesh=mesh,
        compiler_params=pltpu.CompilerParams(needs_layout_passes=False),
        scratch_types=[
            pltpu.VMEM((128,), jnp.int32),
            pltpu.VMEM((128,), jnp.int32),
            pltpu.VMEM((128, D), jnp.float32),
            pltpu.VMEM((16,), jnp.int32),
            pltpu.VMEM((16,), jnp.int32),
            pltpu.VMEM((16, D), jnp.float32),
            pltpu.VMEM_SHARED((_NP, D), jnp.float32),
            pltpu.SemaphoreType.DMA,
        ],
    )
    def k(h_hbm, src_hbm, dst_hbm, out_hbm,
          idx_s, idx_d, rows, idx_s2, idx_d2, rows2, acc_sh, sem):
        cid = lax.axis_index("c")
        sid = lax.axis_index("s")

        def zb(r, carry):
            for kk in range(D // 16):
                rows[r, pl.ds(kk * 16, 16)] = jnp.zeros((16,), jnp.float32)
            return carry
        lax.fori_loop(0, 128, zb, 0)
        r0 = sid * rows_per_tile
        for j in range(rows_per_tile // 128):
            pltpu.sync_copy(rows, acc_sh.at[pl.ds(r0 + j * 128, 128)])
        plsc.subcore_barrier()

        base_e = (cid * 16 + sid) * _EPT

        def chunk(c, carry):
            off = base_e + c * 128
            pltpu.sync_copy(src_hbm.at[pl.ds(off, 128)], idx_s)
            pltpu.sync_copy(dst_hbm.at[pl.ds(off, 128)], idx_d)
            pltpu.async_copy(h_hbm.at[idx_s], rows, sem).wait()
            pltpu.sync_copy(rows, acc_sh.at[idx_d], add=True)
            return carry
        lax.fori_loop(0, _EPT // 128, chunk, 0)

        off = base_e + (_EPT // 128) * 128
        pltpu.sync_copy(src_hbm.at[pl.ds(off, 16)], idx_s2)
        pltpu.sync_copy(dst_hbm.at[pl.ds(off, 16)], idx_d2)
        pltpu.async_copy(h_hbm.at[idx_s2], rows2, sem).wait()
        pltpu.sync_copy(rows2, acc_sh.at[idx_d2], add=True)

        plsc.subcore_barrier()
        pltpu.sync_copy(acc_sh.at[pl.ds(r0, rows_per_tile)],
                        out_hbm.at[pl.ds(cid * _NP + r0, rows_per_tile)])

    return k


_seg_sum128 = _make_seg_sum(_H)


def _edge_gather(h, cef, ref_, tef, src, dst, cts, ctd, rs, rd, et):
    mesh = plsc.VectorSubcoreMesh(core_axis_name="c", subcore_axis_name="s")

    @functools.partial(
        pl.kernel,
        out_type=[
            jax.ShapeDtypeStruct((_E, _H), jnp.float32),
            jax.ShapeDtypeStruct((_E, _H), jnp.float32),
            jax.ShapeDtypeStruct((16, _E), jnp.float32),
            jax.ShapeDtypeStruct((16, _E), jnp.float32),
            jax.ShapeDtypeStruct((16, _E), jnp.float32),
            jax.ShapeDtypeStruct((16, _E), jnp.float32),
            jax.ShapeDtypeStruct((16, _E), jnp.float32),
        ],
        mesh=mesh,
        compiler_params=pltpu.CompilerParams(needs_layout_passes=False),
        scratch_types=[
            pltpu.VMEM((128,), jnp.int32),
            pltpu.VMEM((128, _H), jnp.float32),
            pltpu.VMEM((16, 128), jnp.float32),
            pltpu.VMEM((16,), jnp.int32),
            pltpu.VMEM((16, _H), jnp.float32),
            pltpu.VMEM((4096,), jnp.float32),
            pltpu.VMEM((1024,), jnp.float32),
            pltpu.VMEM((32,), jnp.float32),
            pltpu.SemaphoreType.DMA,
        ],
    )
    def k(h_hbm, ce_hbm, re_hbm, te_hbm,
          src_hbm, dst_hbm, cts_hbm, ctd_hbm, rs_hbm, rd_hbm, et_hbm,
          hs_o, hd_o, c1_o, c2_o, r1_o, r2_o, t_o,
          idxb, bigb, colb, idxb2, bigb2, ce_v, re_v, te_v, sem):
        cid = lax.axis_index("c")
        sid = lax.axis_index("s")
        w = cid * 16 + sid
        pltpu.sync_copy(ce_hbm, ce_v)
        pltpu.sync_copy(re_hbm, re_v)
        pltpu.sync_copy(te_hbm, te_v)
        big_specs = ((src_hbm, hs_o), (dst_hbm, hd_o))
        small_specs = ((ce_v, cts_hbm, c1_o), (ce_v, ctd_hbm, c2_o),
                       (re_v, rs_hbm, r1_o), (re_v, rd_hbm, r2_o),
                       (te_v, et_hbm, t_o))
        nchunks = _E // 128

        def chunk(c, carry):
            cidx = c * 32 + w

            @pl.when(cidx < nchunks)
            def _():
                off = cidx * 128
                for ix, out in big_specs:
                    pltpu.sync_copy(ix.at[pl.ds(off, 128)], idxb)
                    pltpu.async_copy(h_hbm.at[idxb], bigb, sem).wait()
                    pltpu.sync_copy(bigb, out.at[pl.ds(off, 128)])
                for tab_v, ix, out in small_specs:
                    pltpu.sync_copy(ix.at[pl.ds(off, 128)], idxb)

                    def vec(v, carry2):
                        base16 = idxb[pl.ds(v * 16, 16)] * 16
                        for j in range(16):
                            colb[j, pl.ds(v * 16, 16)] = plsc.load_gather(
                                tab_v, [base16 + j])
                        return carry2
                    lax.fori_loop(0, 8, vec, 0)
                    pltpu.sync_copy(colb, out.at[:, pl.ds(off, 128)])
            return carry
        lax.fori_loop(0, (nchunks + 31) // 32, chunk, 0)

    return k(h, cef, ref_, tef, src, dst, cts, ctd, rs, rd, et)


def _edge_head_body(hs_ref, hd_ref, c1_ref, c2_ref, r1_ref, r2_ref, t_ref,
                    es_ref, dat_ref, wa_ref, wb_ref, wc1_ref, wc2_ref,
                    wr1_ref, wr2_ref, wt_ref, ws_ref, b1_ref, w2_ref, b2_ref,
                    awt_ref, ca_ref, swt_ref, cs_ref,
                    dhat_ref, ge_ref, sh_ref, ls_ref, gg_ref):
    dnt = (((0,), (0,)), ((), ()))
    x = jnp.dot(hs_ref[...], wa_ref[...], preferred_element_type=jnp.float32)
    x = x + jnp.dot(hd_ref[...], wb_ref[...], preferred_element_type=jnp.float32)
    for cref, wref in ((c1_ref, wc1_ref), (c2_ref, wc2_ref),
                       (r1_ref, wr1_ref), (r2_ref, wr2_ref), (t_ref, wt_ref)):
        x = x + lax.dot_general(cref[...], wref[...], dnt,
                                preferred_element_type=jnp.float32)
    x = x + jnp.dot(es_ref[...], ws_ref[...], preferred_element_type=jnp.float32)
    x = jax.nn.relu(x + b1_ref[...])
    e = jnp.dot(x, w2_ref[...], preferred_element_type=jnp.float32)
    e = jax.nn.relu(e + b2_ref[...])
    ca = ca_ref[...]
    awt = awt_ref[...]
    l0 = jnp.sum(e * awt[0:1, :], axis=-1, keepdims=True) + ca[:, 0:1]
    l1 = jnp.sum(e * awt[1:2, :], axis=-1, keepdims=True) + ca[:, 1:2]
    l2 = jnp.sum(e * awt[2:3, :], axis=-1, keepdims=True) + ca[:, 2:3]
    m = jnp.maximum(jnp.maximum(l0, l1), l2)
    e0 = jnp.exp(l0 - m)
    e1 = jnp.exp(l1 - m)
    e2 = jnp.exp(l2 - m)
    inv = 1.0 / (e0 + e1 + e2)
    s0 = e0 * inv
    s1 = e1 * inv
    s2 = e2 * inv
    ls = jnp.sum(e * swt_ref[...], axis=-1, keepdims=True) + cs_ref[:, 0:1]
    ls = jnp.clip(ls, -3.0, 3.0)
    da = dat_ref[...]
    ge = (s0 * da[:, 0:1] + s1 * da[:, 1:2] + s2 * da[:, 2:3]) * jnp.exp(ls)
    dhat_ref[...] = jax.nn.relu(ge)
    ge_ref[...] = ge
    sh_ref[:, 0:1] = s0
    sh_ref[:, 1:2] = s1
    sh_ref[:, 2:3] = s2
    ls_ref[...] = ls

    @pl.when(pl.program_id(0) == 0)
    def _():
        gg_ref[...] = jnp.zeros_like(gg_ref)
    gg_ref[...] += jnp.sum(ge) / _E


def _edge_head(hs, hd, c1, c2, r1, r2, tt, es, dat,
               wa, wb, wc1, wc2, wr1, wr2, wt, ws, b1, w2, b2,
               awt, ca, swt, cs):
    grid = _E // _BE
    sm = pl.BlockSpec((16, _BE), lambda i: (0, i))
    w16 = pl.BlockSpec((16, _H), lambda i: (0, 0))
    outs = [
        jax.ShapeDtypeStruct((_E, 1), jnp.float32),
        jax.ShapeDtypeStruct((_E, 1), jnp.float32),
        jax.ShapeDtypeStruct((_E, _K), jnp.float32),
        jax.ShapeDtypeStruct((_E, 1), jnp.float32),
        jax.ShapeDtypeStruct((1, 1), jnp.float32),
    ]
    return pl.pallas_call(
        _edge_head_body,
        grid=(grid,),
        in_specs=[
            pl.BlockSpec((_BE, _H), lambda i: (i, 0)),
            pl.BlockSpec((_BE, _H), lambda i: (i, 0)),
            sm, sm, sm, sm, sm,
            pl.BlockSpec((_BE, 8), lambda i: (i, 0)),
            pl.BlockSpec((_BE, _K), lambda i: (i, 0)),
            pl.BlockSpec((_H, _H), lambda i: (0, 0)),
            pl.BlockSpec((_H, _H), lambda i: (0, 0)),
            w16, w16, w16, w16, w16,
            pl.BlockSpec((8, _H), lambda i: (0, 0)),
            pl.BlockSpec((1, _H), lambda i: (0, 0)),
            pl.BlockSpec((_H, _H), lambda i: (0, 0)),
            pl.BlockSpec((1, _H), lambda i: (0, 0)),
            pl.BlockSpec((_K, _H), lambda i: (0, 0)),
            pl.BlockSpec((1, _K), lambda i: (0, 0)),
            pl.BlockSpec((1, _H), lambda i: (0, 0)),
            pl.BlockSpec((1, 1), lambda i: (0, 0)),
        ],
        out_specs=[
            pl.BlockSpec((_BE, 1), lambda i: (i, 0)),
            pl.BlockSpec((_BE, 1), lambda i: (i, 0)),
            pl.BlockSpec((_BE, _K), lambda i: (i, 0)),
            pl.BlockSpec((_BE, 1), lambda i: (i, 0)),
            pl.BlockSpec((1, 1), lambda i: (0, 0)),
        ],
        out_shape=outs,
    )(hs, hd, c1, c2, r1, r2, tt, es, dat,
      wa, wb, wc1, wc2, wr1, wr2, wt, ws, b1, w2, b2, awt, ca, swt, cs)


def _level_prop(at0, lvlp, src, dst, dh, evf, epi, ratp):
    mesh = plsc.VectorSubcoreMesh(core_axis_name="c", subcore_axis_name="s")
    ept = _E // 16
    rows_per_tile = _NP // 16

    @functools.partial(
        pl.kernel,
        out_type=[
            jax.ShapeDtypeStruct((_NP,), jnp.float32),
            jax.ShapeDtypeStruct((1024,), jnp.float32),
            jax.ShapeDtypeStruct((1024,), jnp.float32),
        ],
        mesh=mesh,
        compiler_params=pltpu.CompilerParams(needs_layout_passes=False),
        scratch_types=[
            pltpu.VMEM((_NP,), jnp.float32),
            pltpu.VMEM((_NP,), jnp.float32),
            pltpu.VMEM((_NP,), jnp.int32),
            pltpu.VMEM((128,), jnp.int32),
            pltpu.VMEM((128,), jnp.int32),
            pltpu.VMEM((128,), jnp.float32),
            pltpu.VMEM((128,), jnp.float32),
            pltpu.VMEM((640,), jnp.float32),
            pltpu.VMEM((640,), jnp.float32),
            pltpu.VMEM((1024,), jnp.int32),
            pltpu.VMEM((1024,), jnp.float32),
            pltpu.VMEM((1024,), jnp.float32),
            pltpu.VMEM((1024,), jnp.float32),
            pltpu.VMEM_SHARED((16, _NP), jnp.float32),
            pltpu.VMEM_SHARED((_NP,), jnp.float32),
            pltpu.SemaphoreType.DMA,
        ],
    )
    def k(at0_hbm, lvl_hbm, src_hbm, dst_hbm, dh_hbm, ev_hbm, epi_hbm, rat_hbm,
          at_out, atep_out, slack_out,
          at_b, cand, lvl_b, ixs, ixd, dhb, evb, tmp, mc,
          epb, ratb, aeb, slb, cand_sh, at_sh, sem):
        cid = lax.axis_index("c")
        sid = lax.axis_index("s")

        if True:
            pltpu.sync_copy(at0_hbm, at_b)
            pltpu.sync_copy(lvl_hbm, lvl_b)
            r0 = sid * rows_per_tile
            base_e = sid * ept

            for lvl_i in range(1, _LMAX):
                def zc(i, carry):
                    cand[pl.ds(i * 16, 16)] = jnp.full((16,), _NEG, jnp.float32)
                    return carry
                lax.fori_loop(0, _NP // 16, zc, 0)

                def do_vec(s_i, d_i, dhv, evv):
                    ld = plsc.load_gather(lvl_b, [d_i])
                    av = plsc.load_gather(at_b, [s_i])
                    msk = (ld == lvl_i) & (evv > 0.5)
                    msg = jnp.where(msk, av + dhv, _NEG)

                    pending0 = jnp.any(plsc.load_gather(cand, [d_i]) < msg)

                    def cond(carry):
                        p, fuel = carry
                        return p & (fuel < 16)

                    def bod(carry):
                        _, fuel = carry
                        cv = plsc.load_gather(cand, [d_i])
                        plsc.store_scatter(cand, [d_i], jnp.maximum(cv, msg))
                        cv2 = plsc.load_gather(cand, [d_i])
                        return jnp.any(cv2 < msg), fuel + 1
                    lax.while_loop(cond, bod, (pending0, jnp.int32(0)))

                def chunk(c, carry):
                    off = base_e + c * 128
                    pltpu.sync_copy(src_hbm.at[pl.ds(off, 128)], ixs)
                    pltpu.sync_copy(dst_hbm.at[pl.ds(off, 128)], ixd)
                    pltpu.sync_copy(dh_hbm.at[pl.ds(off, 128)], dhb)
                    pltpu.sync_copy(ev_hbm.at[pl.ds(off, 128)], evb)
                    for kk in range(8):
                        sl = pl.ds(kk * 16, 16)
                        do_vec(ixs[sl], ixd[sl], dhb[sl], evb[sl])
                    return carry
                lax.fori_loop(0, ept // 128, chunk, 0)

                off = base_e + (ept // 128) * 128
                pltpu.sync_copy(src_hbm.at[pl.ds(off, 32)], ixs.at[pl.ds(0, 32)])
                pltpu.sync_copy(dst_hbm.at[pl.ds(off, 32)], ixd.at[pl.ds(0, 32)])
                pltpu.sync_copy(dh_hbm.at[pl.ds(off, 32)], dhb.at[pl.ds(0, 32)])
                pltpu.sync_copy(ev_hbm.at[pl.ds(off, 32)], evb.at[pl.ds(0, 32)])
                for kk in range(2):
                    sl = pl.ds(kk * 16, 16)
                    do_vec(ixs[sl], ixd[sl], dhb[sl], evb[sl])

                pltpu.sync_copy(cand, cand_sh.at[sid])
                plsc.subcore_barrier()
                pltpu.sync_copy(cand_sh.at[0, pl.ds(r0, rows_per_tile)], mc)
                for s in range(1, 16):
                    pltpu.sync_copy(cand_sh.at[s, pl.ds(r0, rows_per_tile)], tmp)

                    def mg(v, carry):
                        sl = pl.ds(v * 16, 16)
                        mc[sl] = jnp.maximum(mc[sl], tmp[sl])
                        return carry
                    lax.fori_loop(0, rows_per_tile // 16, mg, 0)

                def up(v, carry):
                    slg = pl.ds(r0 + v * 16, 16)
                    sll = pl.ds(v * 16, 16)
                    a = at_b[slg]
                    l = lvl_b[slg]
                    m = mc[sll]
                    at_b[slg] = jnp.where(l == lvl_i, jnp.maximum(a, m), a)
                    return carry
                lax.fori_loop(0, rows_per_tile // 16, up, 0)
                pltpu.sync_copy(at_b.at[pl.ds(r0, rows_per_tile)],
                                at_sh.at[pl.ds(r0, rows_per_tile)])
                plsc.subcore_barrier()
                pltpu.sync_copy(at_sh, at_b)

            @pl.when(cid == 0)
            def _wout():
                pltpu.sync_copy(at_b.at[pl.ds(r0, rows_per_tile)],
                                at_out.at[pl.ds(r0, rows_per_tile)])

            @pl.when((cid == 0) & (sid == 0))
            def _eps():
                pltpu.sync_copy(epi_hbm, epb)
                pltpu.sync_copy(rat_hbm, ratb)

                def ep(v, carry):
                    sl = pl.ds(v * 16, 16)
                    av = plsc.load_gather(at_b, [epb[sl]])
                    aeb[sl] = av
                    slb[sl] = ratb[sl] - av
                    return carry
                lax.fori_loop(0, 1024 // 16, ep, 0)
                pltpu.sync_copy(aeb, atep_out)
                pltpu.sync_copy(slb, slack_out)

    return k(at0, lvlp, src, dst, dh, evf, epi, ratp)


def kernel(pin_static, pin_dyn_anchor, d_anchor, edge_src, edge_dst, edge_type,
           topo_order, node_level, data_mask, edge_valid, source_mask,
           endpoint_ids, rat_true, z_cont, process_id, edge_cell_type_src,
           edge_cell_type_dst, edge_pin_role_src, edge_pin_role_dst,
           edge_fanin_src, edge_fanout_src, edge_fanin_dst, edge_fanout_dst,
           edge_cap_src, edge_cap_dst, edge_scalars_normed, process_embed,
           pvt_proc_embed, vW, vb, tW, tb, Ws0, Wn0, b0, Ws1, Wn1, b1,
           Ws2, Wn2, b2, cell_embed, role_embed, type_embed, eW1, eb1,
           eW2, eb2, aW, ab, sW, sb):
    pid = process_id[0]
    proc_emb = process_embed[pid]
    z_t = jnp.concatenate([proc_emb, z_cont], axis=-1)
    z_pvt = pvt_proc_embed[pid] + z_cont[2:3] @ vW + vb + z_cont[3:4] @ tW + tb

    pin_dyn_flat = jnp.transpose(pin_dyn_anchor, (1, 0, 2)).reshape(_N, _K * 4)
    node_in = jnp.concatenate(
        [pin_static, pin_dyn_flat,
         jnp.broadcast_to(z_t[None, :], (_N, z_t.shape[0])),
         jnp.ones((_N, 1), jnp.float32),
         jnp.zeros((_N, _H - 27), jnp.float32)], axis=-1)
    node_in = jnp.pad(node_in, ((0, _NP - _N), (0, 0)))
    Ws0p = jnp.pad(Ws0, ((0, _H - Ws0.shape[0]), (0, 0)))
    Wn0p = jnp.pad(Wn0, ((0, _H - Wn0.shape[0]), (0, 0)))

    acc0 = _seg_sum128(node_in, edge_src, edge_dst)
    h0, deg2 = _gnn0(node_in, acc0, Ws0p, Wn0p, b0[None, :])
    acc1 = _seg_sum128(h0, edge_src, edge_dst)
    h1 = _gnn12(h0, acc1, deg2, Ws1, Wn1, b1[None, :])
    acc2 = _seg_sum128(h1, edge_src, edge_dst)
    h2 = _gnn12(h1, acc2, deg2, Ws2, Wn2, b2[None, :])

    hs, hd, c1, c2, r1, r2, tt = _edge_gather(
        h2, cell_embed.reshape(-1), role_embed.reshape(-1),
        jnp.pad(type_embed, ((0, 0), (0, 0))).reshape(-1),
        edge_src, edge_dst, edge_cell_type_src, edge_cell_type_dst,
        edge_pin_role_src, edge_pin_role_dst, edge_type)

    es = jnp.pad(edge_scalars_normed, ((0, 0), (0, 2)))
    ca = (z_pvt @ aW[_H:] + ab)[None, :]
    cs = (z_pvt @ sW[_H:] + sb)[None, :]
    awt = jnp.transpose(aW[:_H])
    swt = jnp.transpose(sW[:_H])
    dat = jnp.transpose(d_anchor)

    dhat2, ge2, s_hat, ls2, gg = _edge_head(
        hs, hd, c1, c2, r1, r2, tt, es, dat,
        eW1[0:128], eW1[128:256], eW1[256:272], eW1[272:288],
        eW1[288:304], eW1[304:320], eW1[320:336],
        jnp.pad(eW1[336:342], ((0, 2), (0, 0))),
        eb1[None, :], eW2, eb2[None, :], awt, ca, swt, cs)
    d_hat = dhat2[:, 0]
    g_e = ge2[:, 0]
    log_scale = ls2[:, 0]
    gG = gg[0, 0]

    at0 = jnp.pad(jnp.where(source_mask, jnp.float32(0.0), _NEG),
                  (0, _NP - _N), constant_values=_NEG)
    lvlp = jnp.pad(node_level, (0, _NP - _N))
    evf = edge_valid.astype(jnp.float32)
    epi = jnp.pad(endpoint_ids, (0, 1024 - _P))
    ratp = jnp.pad(rat_true, (0, 1024 - _P))

    at_full, atep_p, slack_p = _level_prop(
        at0, lvlp, edge_src, edge_dst, d_hat, evf, epi, ratp)

    at = at_full[:_N]
    at_ep = atep_p[:_P]
    slack_hat = slack_p[:_P]
    return d_hat, at, at_ep, slack_hat, g_e, gG, s_hat, log_scale


# speedup vs baseline: 8.9807x; 1.7895x over previous
import functools
import jax
import jax.numpy as jnp
from jax import lax
from jax.experimental import pallas as pl
from jax.experimental.pallas import tpu as pltpu
from jax.experimental.pallas import tpu_sc as plsc

_N = 10000
_E = 320000
_K = 3
_H = 128
_P = 1000
_LMAX = 8

_NP = 10240
_BN = 1024
_BE = 2560
_EPT = _E // 32
_NEG = -1e9

_DEG_COL = 26


def _gnn0_body(h_ref, a0_ref, ws_ref, wn_ref, b_ref, o_ref, deg_ref):
    h = h_ref[...]
    acc = a0_ref[...]
    deg = jnp.clip(acc[:, _DEG_COL:_DEG_COL + 1], 1.0, None)
    neigh = acc / deg
    x = jnp.dot(h, ws_ref[...], preferred_element_type=jnp.float32)
    x = x + jnp.dot(neigh, wn_ref[...], preferred_element_type=jnp.float32)
    x = x + b_ref[...]
    m = jnp.mean(x, axis=-1, keepdims=True)
    v = jnp.mean((x - m) * (x - m), axis=-1, keepdims=True)
    o_ref[...] = jax.nn.relu((x - m) / jnp.sqrt(v + 1e-5))
    deg_ref[...] = deg


def _gnn0(h, accflat, Ws, Wn, b):
    din = h.shape[1]
    nblk = _NP // _BN
    return pl.pallas_call(
        _gnn0_body,
        grid=(nblk,),
        in_specs=[
            pl.BlockSpec((_BN, din), lambda i: (i, 0)),
            pl.BlockSpec((_BN, din), lambda i: (i, 0)),
            pl.BlockSpec((din, _H), lambda i: (0, 0)),
            pl.BlockSpec((din, _H), lambda i: (0, 0)),
            pl.BlockSpec((1, _H), lambda i: (0, 0)),
        ],
        out_specs=[
            pl.BlockSpec((_BN, _H), lambda i: (i, 0)),
            pl.BlockSpec((_BN, 1), lambda i: (i, 0)),
        ],
        out_shape=[
            jax.ShapeDtypeStruct((_NP, _H), jnp.float32),
            jax.ShapeDtypeStruct((_NP, 1), jnp.float32),
        ],
    )(h, accflat, Ws, Wn, b)


def _gnn12_body(h_ref, a0_ref, deg_ref, ws_ref, wn_ref, b_ref, o_ref):
    h = h_ref[...]
    neigh = a0_ref[...] / deg_ref[...]
    x = jnp.dot(h, ws_ref[...], preferred_element_type=jnp.float32)
    x = x + jnp.dot(neigh, wn_ref[...], preferred_element_type=jnp.float32)
    x = x + b_ref[...]
    m = jnp.mean(x, axis=-1, keepdims=True)
    v = jnp.mean((x - m) * (x - m), axis=-1, keepdims=True)
    y = jax.nn.relu((x - m) / jnp.sqrt(v + 1e-5))
    o_ref[...] = 0.5 * y + 0.5 * h


def _gnn12(h, accflat, deg2, Ws, Wn, b):
    nblk = _NP // _BN
    return pl.pallas_call(
        _gnn12_body,
        grid=(nblk,),
        in_specs=[
            pl.BlockSpec((_BN, _H), lambda i: (i, 0)),
            pl.BlockSpec((_BN, _H), lambda i: (i, 0)),
            pl.BlockSpec((_BN, 1), lambda i: (i, 0)),
            pl.BlockSpec((_H, _H), lambda i: (0, 0)),
            pl.BlockSpec((_H, _H), lambda i: (0, 0)),
            pl.BlockSpec((1, _H), lambda i: (0, 0)),
        ],
        out_specs=pl.BlockSpec((_BN, _H), lambda i: (i, 0)),
        out_shape=jax.ShapeDtypeStruct((_NP, _H), jnp.float32),
    )(h, accflat, deg2, Ws, Wn, b)


def _make_seg_sum(D):
    mesh = plsc.VectorSubcoreMesh(core_axis_name="c", subcore_axis_name="s")
    rows_per_tile = _NP // 16

    nrows = _E // 128
    rpt = 160
    half = _NP // 2
    hacc = half + 8
    rows_out = half // 16

    @functools.partial(
        pl.kernel,
        out_type=jax.ShapeDtypeStruct((_NP, D), jnp.float32),
        mesh=mesh,
        compiler_params=pltpu.CompilerParams(needs_layout_passes=False),
        scratch_types=[
            pltpu.VMEM((rpt, 128), jnp.int32),
            pltpu.VMEM((rpt, 128), jnp.int32),
            pltpu.VMEM((128, D), jnp.float32),
            pltpu.VMEM((128, D), jnp.float32),
            pltpu.VMEM_SHARED((hacc, D), jnp.float32),
            pltpu.SemaphoreType.DMA,
            pltpu.SemaphoreType.DMA,
        ],
    )
    def k(h_hbm, src2_hbm, dsel2_hbm, out_hbm,
          idxs2, idxd2, rows, rows1, acc_sh, sem, sem1):
        cid = lax.axis_index("c")
        sid = lax.axis_index("s")

        def zb(r, carry):
            for kk in range(D // 16):
                rows[r, pl.ds(kk * 16, 16)] = jnp.zeros((16,), jnp.float32)
            return carry
        lax.fori_loop(0, 128, zb, 0)
        r0 = sid * rows_out
        for j in range(rows_out // 128):
            pltpu.sync_copy(rows, acc_sh.at[pl.ds(r0 + j * 128, 128)])
        pltpu.sync_copy(rows.at[pl.ds(0, 64)],
                        acc_sh.at[pl.ds(r0 + 256, 64)])

        @pl.when(sid == 0)
        def _zdump():
            pltpu.sync_copy(rows.at[pl.ds(0, 8)], acc_sh.at[pl.ds(half, 8)])
        plsc.subcore_barrier()

        pltpu.sync_copy(src2_hbm.at[pl.ds(sid * rpt, rpt)], idxs2)
        pltpu.sync_copy(dsel2_hbm.at[cid, pl.ds(sid * rpt, rpt)], idxd2)

        def chunk(c, carry):
            @pl.when(sid * rpt + 2 * c < nrows)
            def _():
                g0 = pltpu.async_copy(h_hbm.at[idxs2.at[2 * c]], rows, sem)
                g1 = pltpu.async_copy(h_hbm.at[idxs2.at[2 * c + 1]], rows1, sem1)
                g0.wait()
                pltpu.sync_copy(rows, acc_sh.at[idxd2.at[2 * c]], add=True)
                g1.wait()
                pltpu.sync_copy(rows1, acc_sh.at[idxd2.at[2 * c + 1]], add=True)
            return carry
        lax.fori_loop(0, rpt // 2, chunk, 0)

        plsc.subcore_barrier()
        pltpu.sync_copy(acc_sh.at[pl.ds(r0, rows_out)],
                        out_hbm.at[pl.ds(cid * half + r0, rows_out)])

    return k


_seg_sum128 = _make_seg_sum(_H)


def _edge_gather(h, tf, src2, dst2, pidx2):
    mesh = plsc.VectorSubcoreMesh(core_axis_name="c", subcore_axis_name="s")
    nrows = _E // 128
    rpt = 80

    @functools.partial(
        pl.kernel,
        out_type=[
            jax.ShapeDtypeStruct((_E, _H), jnp.float32),
            jax.ShapeDtypeStruct((_E, _H), jnp.float32),
            jax.ShapeDtypeStruct((80, _E), jnp.float32),
        ],
        mesh=mesh,
        compiler_params=pltpu.CompilerParams(needs_layout_passes=False),
        scratch_types=[
            pltpu.VMEM((rpt, 128), jnp.int32),
            pltpu.VMEM((rpt, 128), jnp.int32),
            pltpu.VMEM((5 * rpt, 128), jnp.int32),
            pltpu.VMEM((128, _H), jnp.float32),
            pltpu.VMEM((128, _H), jnp.float32),
            pltpu.VMEM((80, 128), jnp.float32),
            pltpu.VMEM((5152,), jnp.float32),
            pltpu.SemaphoreType.DMA,
            pltpu.SemaphoreType.DMA,
        ],
    )
    def k(h_hbm, tf_hbm, src2_hbm, dst2_hbm, pidx2_hbm,
          hs_o, hd_o, emb_o,
          idxs2, idxd2, pxb, bigb_s, bigb_d, colb5, tf_v, sem, sem1):
        cid = lax.axis_index("c")
        sid = lax.axis_index("s")
        w = cid * 16 + sid
        pltpu.sync_copy(tf_hbm, tf_v)
        pltpu.sync_copy(src2_hbm.at[pl.ds(w * rpt, rpt)], idxs2)
        pltpu.sync_copy(dst2_hbm.at[pl.ds(w * rpt, rpt)], idxd2)
        for t in range(5):
            pltpu.sync_copy(pidx2_hbm.at[pl.ds(t * 32 * rpt + w * rpt, rpt)],
                            pxb.at[pl.ds(t * rpt, rpt)])

        def chunk(c, carry):
            @pl.when(w * rpt + c < nrows)
            def _():
                off = (w * rpt + c) * 128
                g0 = pltpu.async_copy(h_hbm.at[idxs2.at[c]], bigb_s, sem)
                g1 = pltpu.async_copy(h_hbm.at[idxd2.at[c]], bigb_d, sem1)
                for t in range(5):
                    def vec(v, carry2, t=t):
                        fidx = pxb[t * rpt + c, pl.ds(v * 16, 16)]
                        for j in range(16):
                            colb5[t * 16 + j, pl.ds(v * 16, 16)] = \
                                plsc.load_gather(tf_v, [fidx + j])
                        return carry2
                    lax.fori_loop(0, 8, vec, 0)
                pltpu.sync_copy(colb5, emb_o.at[:, pl.ds(off, 128)])
                g0.wait()
                pltpu.sync_copy(bigb_s, hs_o.at[pl.ds(off, 128)])
                g1.wait()
                pltpu.sync_copy(bigb_d, hd_o.at[pl.ds(off, 128)])
            return carry
        lax.fori_loop(0, rpt, chunk, 0)

    return k(h, tf, src2, dst2, pidx2)


def _edge_head_body(hs_ref, hd_ref, emb_ref,
                    es_ref, dat_ref, wa_ref, wb_ref, we_ref, ws_ref,
                    b1_ref, w2_ref, b2_ref,
                    awt_ref, ca_ref, swt_ref, cs_ref,
                    dhat_ref, ge_ref, sh_ref, ls_ref, gg_ref):
    dnt = (((0,), (0,)), ((), ()))
    x = jnp.dot(hs_ref[...], wa_ref[...], preferred_element_type=jnp.float32)
    x = x + jnp.dot(hd_ref[...], wb_ref[...], preferred_element_type=jnp.float32)
    x = x + lax.dot_general(emb_ref[...], we_ref[...], dnt,
                            preferred_element_type=jnp.float32)
    x = x + jnp.dot(es_ref[...], ws_ref[...], preferred_element_type=jnp.float32)
    x = jax.nn.relu(x + b1_ref[...])
    e = jnp.dot(x, w2_ref[...], preferred_element_type=jnp.float32)
    e = jax.nn.relu(e + b2_ref[...])
    ca = ca_ref[...]
    awt = awt_ref[...]
    l0 = jnp.sum(e * awt[0:1, :], axis=-1, keepdims=True) + ca[:, 0:1]
    l1 = jnp.sum(e * awt[1:2, :], axis=-1, keepdims=True) + ca[:, 1:2]
    l2 = jnp.sum(e * awt[2:3, :], axis=-1, keepdims=True) + ca[:, 2:3]
    m = jnp.maximum(jnp.maximum(l0, l1), l2)
    e0 = jnp.exp(l0 - m)
    e1 = jnp.exp(l1 - m)
    e2 = jnp.exp(l2 - m)
    inv = 1.0 / (e0 + e1 + e2)
    s0 = e0 * inv
    s1 = e1 * inv
    s2 = e2 * inv
    ls = jnp.sum(e * swt_ref[...], axis=-1, keepdims=True) + cs_ref[:, 0:1]
    ls = jnp.clip(ls, -3.0, 3.0)
    da = dat_ref[...]
    ge = (s0 * da[:, 0:1] + s1 * da[:, 1:2] + s2 * da[:, 2:3]) * jnp.exp(ls)
    dhat_ref[...] = jax.nn.relu(ge)
    ge_ref[...] = ge
    sh_ref[:, 0:1] = s0
    sh_ref[:, 1:2] = s1
    sh_ref[:, 2:3] = s2
    ls_ref[...] = ls

    @pl.when(pl.program_id(0) == 0)
    def _():
        gg_ref[...] = jnp.zeros_like(gg_ref)
    gg_ref[...] += jnp.sum(ge) / _E


def _edge_head(hs, hd, emb, es, dat,
               wa, wb, we, ws, b1, w2, b2,
               awt, ca, swt, cs):
    grid = _E // _BE
    outs = [
        jax.ShapeDtypeStruct((_E, 1), jnp.float32),
        jax.ShapeDtypeStruct((_E, 1), jnp.float32),
        jax.ShapeDtypeStruct((_E, _K), jnp.float32),
        jax.ShapeDtypeStruct((_E, 1), jnp.float32),
        jax.ShapeDtypeStruct((1, 1), jnp.float32),
    ]
    return pl.pallas_call(
        _edge_head_body,
        grid=(grid,),
        in_specs=[
            pl.BlockSpec((_BE, _H), lambda i: (i, 0)),
            pl.BlockSpec((_BE, _H), lambda i: (i, 0)),
            pl.BlockSpec((80, _BE), lambda i: (0, i)),
            pl.BlockSpec((_BE, 8), lambda i: (i, 0)),
            pl.BlockSpec((_BE, _K), lambda i: (i, 0)),
            pl.BlockSpec((_H, _H), lambda i: (0, 0)),
            pl.BlockSpec((_H, _H), lambda i: (0, 0)),
            pl.BlockSpec((80, _H), lambda i: (0, 0)),
            pl.BlockSpec((8, _H), lambda i: (0, 0)),
            pl.BlockSpec((1, _H), lambda i: (0, 0)),
            pl.BlockSpec((_H, _H), lambda i: (0, 0)),
            pl.BlockSpec((1, _H), lambda i: (0, 0)),
            pl.BlockSpec((_K, _H), lambda i: (0, 0)),
            pl.BlockSpec((1, _K), lambda i: (0, 0)),
            pl.BlockSpec((1, _H), lambda i: (0, 0)),
            pl.BlockSpec((1, 1), lambda i: (0, 0)),
        ],
        out_specs=[
            pl.BlockSpec((_BE, 1), lambda i: (i, 0)),
            pl.BlockSpec((_BE, 1), lambda i: (i, 0)),
            pl.BlockSpec((_BE, _K), lambda i: (i, 0)),
            pl.BlockSpec((_BE, 1), lambda i: (i, 0)),
            pl.BlockSpec((1, 1), lambda i: (0, 0)),
        ],
        out_shape=outs,
    )(hs, hd, emb, es, dat, wa, wb, we, ws, b1, w2, b2, awt, ca, swt, cs)


def _level_prop(at0, lvlp, src, dst, dh, epi, ratp):
    mesh = plsc.VectorSubcoreMesh(core_axis_name="c", subcore_axis_name="s")
    ept = _E // 16
    rows_per_tile = _NP // 16

    @functools.partial(
        pl.kernel,
        out_type=[
            jax.ShapeDtypeStruct((_NP,), jnp.float32),
            jax.ShapeDtypeStruct((1024,), jnp.float32),
            jax.ShapeDtypeStruct((1024,), jnp.float32),
        ],
        mesh=mesh,
        compiler_params=pltpu.CompilerParams(needs_layout_passes=False),
        scratch_types=[
            pltpu.VMEM((_NP,), jnp.float32),
            pltpu.VMEM((_NP,), jnp.float32),
            pltpu.VMEM((_NP,), jnp.int32),
            pltpu.VMEM((_E // 16,), jnp.int32),
            pltpu.VMEM((_E // 16,), jnp.int32),
            pltpu.VMEM((_E // 16,), jnp.float32),
            pltpu.VMEM((_NP,), jnp.float32),
            pltpu.VMEM((1024,), jnp.int32),
            pltpu.VMEM((1024,), jnp.float32),
            pltpu.VMEM((1024,), jnp.float32),
            pltpu.VMEM((1024,), jnp.float32),
            pltpu.VMEM_SHARED((8, _NP), jnp.float32),
            pltpu.VMEM_SHARED((_NP,), jnp.float32),
            pltpu.SemaphoreType.DMA,
        ],
    )
    def k(at0_hbm, lvl_hbm, src_hbm, dst_hbm, dh_hbm, epi_hbm, rat_hbm,
          at_out, atep_out, slack_out,
          at_b, cand, lvl_b, ixs, ixd, dhb, tmpf,
          epb, ratb, aeb, slb, cand_sh, at_sh, sem):
        cid = lax.axis_index("c")
        sid = lax.axis_index("s")

        if True:
            pltpu.sync_copy(at0_hbm, at_b)
            pltpu.sync_copy(lvl_hbm, lvl_b)
            r0 = sid * rows_per_tile
            base_e = sid * ept
            pltpu.sync_copy(src_hbm.at[pl.ds(base_e, ept)], ixs)
            pltpu.sync_copy(dst_hbm.at[pl.ds(base_e, ept)], ixd)
            pltpu.sync_copy(dh_hbm.at[pl.ds(base_e, ept)], dhb)

            for lvl_i in range(1, _LMAX):
                def zc(i, carry):
                    cand[pl.ds(i * 16, 16)] = jnp.full((16,), _NEG, jnp.float32)
                    return carry
                lax.fori_loop(0, _NP // 16, zc, 0)

                def do_vec(s_i, d_i, dhv):
                    ld = plsc.load_gather(lvl_b, [d_i])
                    av = plsc.load_gather(at_b, [s_i])
                    msg = jnp.where(ld == lvl_i, av + dhv, _NEG)

                    pending0 = jnp.any(plsc.load_gather(cand, [d_i]) < msg)

                    def cond(carry):
                        p, fuel = carry
                        return p & (fuel < 16)

                    def bod(carry):
                        _, fuel = carry
                        cv = plsc.load_gather(cand, [d_i])
                        plsc.store_scatter(cand, [d_i], jnp.maximum(cv, msg))
                        cv2 = plsc.load_gather(cand, [d_i])
                        return jnp.any(cv2 < msg), fuel + 1
                    lax.while_loop(cond, bod, (pending0, jnp.int32(0)))

                def vecloop(v, carry):
                    sl = pl.ds(v * 16, 16)
                    do_vec(ixs[sl], ixd[sl], dhb[sl])
                    return carry
                lax.fori_loop(0, ept // 16, vecloop, 0)

                for half in (8, 4, 2, 1):
                    @pl.when((sid >= half) & (sid < 2 * half))
                    def _pub(half=half):
                        pltpu.sync_copy(cand, cand_sh.at[sid - half])
                    plsc.subcore_barrier()

                    @pl.when(sid < half)
                    def _mrg(half=half):
                        pltpu.sync_copy(cand_sh.at[sid], tmpf)

                        def mx(v, carry):
                            sl = pl.ds(v * 16, 16)
                            cand[sl] = jnp.maximum(cand[sl], tmpf[sl])
                            return carry
                        lax.fori_loop(0, _NP // 16, mx, 0)
                    plsc.subcore_barrier()

                @pl.when(sid == 0)
                def _pubm():
                    pltpu.sync_copy(cand, cand_sh.at[0])
                plsc.subcore_barrier()
                pltpu.sync_copy(cand_sh.at[0, pl.ds(r0, rows_per_tile)],
                                tmpf.at[pl.ds(0, rows_per_tile)])

                def up(v, carry):
                    sll = pl.ds(v * 16, 16)
                    slg = pl.ds(r0 + v * 16, 16)
                    a = at_b[slg]
                    l = lvl_b[slg]
                    m = tmpf[sll]
                    at_b[slg] = jnp.where(l == lvl_i, jnp.maximum(a, m), a)
                    return carry
                lax.fori_loop(0, rows_per_tile // 16, up, 0)
                pltpu.sync_copy(at_b.at[pl.ds(r0, rows_per_tile)],
                                at_sh.at[pl.ds(r0, rows_per_tile)])
                plsc.subcore_barrier()
                pltpu.sync_copy(at_sh, at_b)

            @pl.when(cid == 0)
            def _wout():
                pltpu.sync_copy(at_b.at[pl.ds(r0, rows_per_tile)],
                                at_out.at[pl.ds(r0, rows_per_tile)])

            @pl.when((cid == 0) & (sid == 0))
            def _eps():
                pltpu.sync_copy(epi_hbm, epb)
                pltpu.sync_copy(rat_hbm, ratb)

                def ep(v, carry):
                    sl = pl.ds(v * 16, 16)
                    av = plsc.load_gather(at_b, [epb[sl]])
                    aeb[sl] = av
                    slb[sl] = ratb[sl] - av
                    return carry
                lax.fori_loop(0, 1024 // 16, ep, 0)
                pltpu.sync_copy(aeb, atep_out)
                pltpu.sync_copy(slb, slack_out)

    return k(at0, lvlp, src, dst, dh, epi, ratp)


def kernel(pin_static, pin_dyn_anchor, d_anchor, edge_src, edge_dst, edge_type,
           topo_order, node_level, data_mask, edge_valid, source_mask,
           endpoint_ids, rat_true, z_cont, process_id, edge_cell_type_src,
           edge_cell_type_dst, edge_pin_role_src, edge_pin_role_dst,
           edge_fanin_src, edge_fanout_src, edge_fanin_dst, edge_fanout_dst,
           edge_cap_src, edge_cap_dst, edge_scalars_normed, process_embed,
           pvt_proc_embed, vW, vb, tW, tb, Ws0, Wn0, b0, Ws1, Wn1, b1,
           Ws2, Wn2, b2, cell_embed, role_embed, type_embed, eW1, eb1,
           eW2, eb2, aW, ab, sW, sb):
    pid = process_id[0]
    proc_emb = process_embed[pid]
    z_t = jnp.concatenate([proc_emb, z_cont], axis=-1)
    z_pvt = pvt_proc_embed[pid] + z_cont[2:3] @ vW + vb + z_cont[3:4] @ tW + tb

    pin_dyn_flat = jnp.transpose(pin_dyn_anchor, (1, 0, 2)).reshape(_N, _K * 4)
    node_in = jnp.concatenate(
        [pin_static, pin_dyn_flat,
         jnp.broadcast_to(z_t[None, :], (_N, z_t.shape[0])),
         jnp.ones((_N, 1), jnp.float32),
         jnp.zeros((_N, _H - 27), jnp.float32)], axis=-1)
    node_in = jnp.pad(node_in, ((0, _NP - _N), (0, 0)))
    Ws0p = jnp.pad(Ws0, ((0, _H - Ws0.shape[0]), (0, 0)))
    Wn0p = jnp.pad(Wn0, ((0, _H - Wn0.shape[0]), (0, 0)))

    ep_pad = 32 * 80 * 128 - _E
    src2 = jnp.pad(edge_src, (0, ep_pad)).reshape(-1, 128)
    dst2 = jnp.pad(edge_dst, (0, ep_pad)).reshape(-1, 128)
    half = _NP // 2
    dsel2 = jnp.stack([jnp.where(dst2 < half, dst2, half),
                       jnp.where(dst2 >= half, dst2 - half, half)])
    acc0 = _seg_sum128(node_in, src2, dsel2)
    h0, deg2 = _gnn0(node_in, acc0, Ws0p, Wn0p, b0[None, :])
    acc1 = _seg_sum128(h0, src2, dsel2)
    h1 = _gnn12(h0, acc1, deg2, Ws1, Wn1, b1[None, :])
    acc2 = _seg_sum128(h1, src2, dsel2)
    h2 = _gnn12(h1, acc2, deg2, Ws2, Wn2, b2[None, :])

    tf = jnp.concatenate([cell_embed.reshape(-1), role_embed.reshape(-1),
                          type_embed.reshape(-1)])
    pidx = jnp.concatenate([
        jnp.pad(t, (0, ep_pad)) for t in (
            edge_cell_type_src * 16, edge_cell_type_dst * 16,
            4096 + edge_pin_role_src * 16, 4096 + edge_pin_role_dst * 16,
            5120 + edge_type * 16)]).reshape(-1, 128)

    hs, hd, emb = _edge_gather(h2, tf, src2, dst2, pidx)

    es = jnp.pad(edge_scalars_normed, ((0, 0), (0, 2)))
    ca = (z_pvt @ aW[_H:] + ab)[None, :]
    cs = (z_pvt @ sW[_H:] + sb)[None, :]
    awt = jnp.transpose(aW[:_H])
    swt = jnp.transpose(sW[:_H])
    dat = jnp.transpose(d_anchor)

    dhat2, ge2, s_hat, ls2, gg = _edge_head(
        hs, hd, emb, es, dat,
        eW1[0:128], eW1[128:256], eW1[256:336],
        jnp.pad(eW1[336:342], ((0, 2), (0, 0))),
        eb1[None, :], eW2, eb2[None, :], awt, ca, swt, cs)
    d_hat = dhat2[:, 0]
    g_e = ge2[:, 0]
    log_scale = ls2[:, 0]
    gG = gg[0, 0]

    at0 = jnp.pad(jnp.where(source_mask, jnp.float32(0.0), _NEG),
                  (0, _NP - _N), constant_values=_NEG)
    lvlp = jnp.pad(node_level, (0, _NP - _N))
    dhm = jnp.where(edge_valid, d_hat, _NEG)
    epi = jnp.pad(endpoint_ids, (0, 1024 - _P))
    ratp = jnp.pad(rat_true, (0, 1024 - _P))

    at_full, atep_p, slack_p = _level_prop(
        at0, lvlp, edge_src, edge_dst, dhm, epi, ratp)

    at = at_full[:_N]
    at_ep = atep_p[:_P]
    slack_hat = slack_p[:_P]
    return d_hat, at, at_ep, slack_hat, g_e, gG, s_hat, log_scale
